# Initial kernel scaffold; baseline (speedup 1.0000x reference)
import jax
import jax.numpy as jnp
from jax.experimental import pallas as pl


def kernel(x, W, b):
    raise NotImplementedError("write your pallas kernel here")


# speedup vs baseline: 32.7570x; 32.7570x over previous
import jax
import jax.numpy as jnp
from jax.experimental import pallas as pl
from jax.experimental.pallas import tpu as pltpu

_B = 4096
_F = 64
_N = 32768
_K = 1024
_R = 128

def _block_kernel(x_ref, w_ref, b_ref, out_ref, key_ref):
    y = jax.lax.dot_general(
        x_ref[...], w_ref[...],
        dimension_numbers=(((1,), (0,)), ((), ())),
        preferred_element_type=jnp.float32,
    ) + b_ref[...]

    bits = jax.lax.bitcast_convert_type(y, jnp.int32)
    int_min = jnp.int32(-2147483648)
    key = jnp.where(bits >= 0, bits, int_min - bits)
    key_ref[...] = key
    out_ref[...] = y

    lo = jnp.min(key, axis=1, keepdims=True)
    hi = jnp.max(key, axis=1, keepdims=True)

    def body(_, carry):
        lo, hi = carry
        mid = (lo >> 1) + (hi >> 1) + 1
        cnt = jnp.sum((key_ref[...] >= mid).astype(jnp.int32), axis=1,
                      keepdims=True)
        ge = cnt >= _K
        return jnp.where(ge, mid, lo), jnp.where(ge, hi, mid - 1)

    lo, hi = jax.lax.fori_loop(0, 32, body, (lo, hi))

    yv = out_ref[...]
    keep = key_ref[...] >= lo
    out_ref[...] = jnp.where(keep, jnp.where(yv > 0, yv, 0.1 * yv), 0.0)


def kernel(x, W, b):
    b2 = b.reshape(1, _N)
    Wt = W.T
    grid = _B // _R
    return pl.pallas_call(
        _block_kernel,
        grid=(grid,),
        in_specs=[
            pl.BlockSpec((_R, _F), lambda i: (i, 0)),
            pl.BlockSpec((_F, _N), lambda i: (0, 0)),
            pl.BlockSpec((1, _N), lambda i: (0, 0)),
        ],
        out_specs=pl.BlockSpec((_R, _N), lambda i: (i, 0)),
        out_shape=jax.ShapeDtypeStruct((_B, _N), jnp.float32),
        scratch_shapes=[pltpu.VMEM((_R, _N), jnp.int32)],
    )(x, Wt, b2)


# speedup vs baseline: 55.0117x; 1.6794x over previous
import jax
import jax.numpy as jnp
from jax.experimental import pallas as pl
from jax.experimental.pallas import tpu as pltpu

_B = 4096
_F = 64
_N = 32768
_K = 1024
_R = 128

_Z_QUANTILE = 1.8627
_PROBE_DELTA = 0.08


def _fkey(f):
    bits = jax.lax.bitcast_convert_type(f, jnp.int32)
    return jnp.where(bits >= 0, bits, jnp.int32(-2147483648) - bits)


def _fval(k):
    bits = jnp.where(k >= 0, k, jnp.int32(-2147483648) - k)
    return jax.lax.bitcast_convert_type(bits, jnp.float32)


def _block_kernel(x_ref, w_ref, b_ref, out_ref):
    y = jax.lax.dot_general(
        x_ref[...], w_ref[...],
        dimension_numbers=(((1,), (0,)), ((), ())),
        preferred_element_type=jnp.float32,
    ) + b_ref[...]
    out_ref[...] = y

    n = jnp.float32(_N)
    mu = jnp.sum(y, axis=1, keepdims=True) / n
    var = jnp.maximum(jnp.sum(y * y, axis=1, keepdims=True) / n - mu * mu,
                      0.0)
    sig = jnp.sqrt(var)
    t_est = mu + jnp.float32(_Z_QUANTILE) * sig
    k_est = _fkey(t_est)
    k_lo_probe = _fkey(t_est - jnp.float32(_PROBE_DELTA) * sig)
    k_hi_probe = _fkey(t_est + jnp.float32(_PROBE_DELTA) * sig)

    lo = _fkey(jnp.min(y, axis=1, keepdims=True))
    hi = _fkey(jnp.max(y, axis=1, keepdims=True))

    def cond(carry):
        it, lo, hi = carry
        return jnp.logical_and(it < 40, jnp.any(lo < hi))

    def body(carry):
        it, lo, hi = carry
        mid_arith = (lo >> 1) + (hi >> 1) + 1
        probe2 = jnp.where(lo >= k_est, k_hi_probe, k_lo_probe)
        mid = jnp.where(it == 0, k_est,
                        jnp.where(it == 1, probe2, mid_arith))
        mid = jnp.clip(mid, lo + 1, hi)
        cnt = jnp.sum((out_ref[...] >= _fval(mid)).astype(jnp.int32),
                      axis=1, keepdims=True)
        upd = lo < hi
        eq = cnt == _K
        ge = cnt >= _K
        new_lo = jnp.where(eq, mid, jnp.where(ge, mid, lo))
        new_hi = jnp.where(eq, mid, jnp.where(ge, hi, mid - 1))
        return (it + 1,
                jnp.where(upd, new_lo, lo),
                jnp.where(upd, new_hi, hi))

    _, lo, _ = jax.lax.while_loop(cond, body, (jnp.int32(0), lo, hi))

    yv = out_ref[...]
    keep = yv >= _fval(lo)
    out_ref[...] = jnp.where(keep, jnp.where(yv > 0, yv, 0.1 * yv), 0.0)


def kernel(x, W, b):
    b2 = b.reshape(1, _N)
    Wt = W.T
    grid = _B // _R
    return pl.pallas_call(
        _block_kernel,
        grid=(grid,),
        in_specs=[
            pl.BlockSpec((_R, _F), lambda i: (i, 0)),
            pl.BlockSpec((_F, _N), lambda i: (0, 0)),
            pl.BlockSpec((1, _N), lambda i: (0, 0)),
        ],
        out_specs=pl.BlockSpec((_R, _N), lambda i: (i, 0)),
        out_shape=jax.ShapeDtypeStruct((_B, _N), jnp.float32),
    )(x, Wt, b2)
